# baseline (speedup 1.0000x reference)
import functools

import jax
import jax.numpy as jnp
from jax import lax
from jax.experimental import pallas as pl
from jax.experimental.pallas import tpu as pltpu
from jax.experimental.pallas import tpu_sc as plsc

_K = 0.05
_EPS = 0.3

_NW = 32
_CH_ROWS = 2


def _tc_body(x_ref, y_ref, z_ref, out_ref, acc_ref):
    step = pl.program_id(0)
    nsteps = pl.num_programs(0)

    @pl.when(step == 0)
    def _init():
        acc_ref[0] = 0.0
        acc_ref[1] = 0.0

    xb = x_ref[...]
    yb = y_ref[...][None, :]
    rows = lax.broadcasted_iota(jnp.int32, xb.shape, 0)
    onehot = rows == yb
    target = jnp.sum(jnp.where(onehot, xb, 0.0), axis=0)
    rest_max = jnp.max(jnp.where(onehot, -jnp.inf, xb), axis=0)
    acc_ref[0] += jnp.sum(jnp.maximum(target - rest_max + _K, 0.0))
    acc_ref[1] += jnp.sum(z_ref[...])

    @pl.when(step == nsteps - 1)
    def _fini():
        out_ref[0, 0] = acc_ref[0]
        out_ref[0, 1] = acc_ref[1]


def _tc_part(x, y, z):
    B, C = x.shape
    BB = 2048
    grid = B // BB
    xt = x.T
    return pl.pallas_call(
        _tc_body,
        grid=(grid,),
        in_specs=[
            pl.BlockSpec((C, BB), lambda i: (0, i)),
            pl.BlockSpec((BB,), lambda i: (i,)),
            pl.BlockSpec((BB,), lambda i: (i,)),
        ],
        out_specs=pl.BlockSpec(
            (1, 2), lambda i: (0, 0), memory_space=pltpu.SMEM
        ),
        out_shape=jax.ShapeDtypeStruct((1, 2), jnp.float32),
        scratch_shapes=[pltpu.SMEM((2,), jnp.float32)],
    )(xt, y.astype(jnp.int32), z)


def _sc_body(nchunks, dt_ref, out_ref, buf0, buf1, stage, sem0, sem1):
    w = lax.axis_index("s") * 2 + lax.axis_index("c")
    bufs = (buf0, buf1)
    sems = (sem0, sem1)
    total = -(-nchunks // _NW)

    def _src(k):
        cid = k * _NW + w
        row = jnp.where(cid < nchunks, cid, w) * _CH_ROWS
        return dt_ref.at[pl.ds(row, _CH_ROWS)]

    descs = [None, None]
    descs[0] = pltpu.make_async_copy(_src(0), buf0, sem0)
    descs[0].start()

    accs = [jnp.full((16,), -jnp.inf, jnp.float32) for _ in range(4)]

    def _reduce(buf, accs):
        def body(i, accs):
            a = list(accs)
            for r in range(_CH_ROWS):
                for u in range(8):
                    sl = buf[r, pl.ds(i * 128 + u * 16, 16)]
                    j = (r * 8 + u) % 4
                    a[j] = jnp.maximum(a[j], sl)
            return tuple(a)
        return lax.fori_loop(0, 16384 // 128, body, tuple(accs))

    for k in range(total):
        cur = k % 2
        if k + 1 < total:
            nxt = (k + 1) % 2
            descs[nxt] = pltpu.make_async_copy(_src(k + 1), bufs[nxt], sems[nxt])
            descs[nxt].start()
        descs[cur].wait()
        accs = list(_reduce(bufs[cur], accs))

    acc = jnp.maximum(jnp.maximum(accs[0], accs[1]),
                      jnp.maximum(accs[2], accs[3]))
    stage[...] = acc
    pltpu.sync_copy(stage, out_ref.at[w])


def _sc_delta_max(delta):
    dt = delta.T
    D, B = dt.shape
    nchunks = D // _CH_ROWS
    mesh = plsc.VectorSubcoreMesh(core_axis_name="c", subcore_axis_name="s")
    kern = functools.partial(
        pl.kernel,
        mesh=mesh,
        out_type=jax.ShapeDtypeStruct((_NW, 16), jnp.float32),
        scratch_types=[
            pltpu.VMEM((_CH_ROWS, B), jnp.float32),
            pltpu.VMEM((_CH_ROWS, B), jnp.float32),
            pltpu.VMEM((16,), jnp.float32),
            pltpu.SemaphoreType.DMA,
            pltpu.SemaphoreType.DMA,
        ],
    )(functools.partial(_sc_body, nchunks))
    return kern(dt)


def kernel(x, delta, y, z):
    B = x.shape[0]
    partials = _tc_part(x, y, z)
    dmax_parts = _sc_delta_max(delta)
    dmax = jnp.max(dmax_parts)
    b = jnp.float32(B)
    return partials[0, 0] / b + (partials[0, 1] / b) * (_EPS + dmax)


# speedup vs baseline: 1.3732x; 1.3732x over previous
import jax
import jax.numpy as jnp
from jax import lax
from jax.experimental import pallas as pl
from jax.experimental.pallas import tpu as pltpu

_K = 0.05
_EPS = 0.3
_BB = 2048


def _body(x_ref, y_ref, d_ref, z_ref, out_ref, acc_ref):
    step = pl.program_id(0)
    nsteps = pl.num_programs(0)

    @pl.when(step == 0)
    def _init():
        acc_ref[0] = 0.0
        acc_ref[1] = 0.0
        acc_ref[2] = -jnp.inf

    xb = x_ref[...]
    yb = y_ref[pl.ds(step * _BB, _BB)][None, :]
    rows = lax.broadcasted_iota(jnp.int32, xb.shape, 0)
    onehot = rows == yb
    target = jnp.sum(jnp.where(onehot, xb, 0.0), axis=0)
    rest_max = jnp.max(jnp.where(onehot, -jnp.inf, xb), axis=0)
    relu_sum = jnp.sum(jnp.maximum(target - rest_max + _K, 0.0))

    acc_ref[0] += relu_sum
    acc_ref[2] = jnp.maximum(acc_ref[2], jnp.max(d_ref[...]))

    @pl.when(step == nsteps - 1)
    def _fini():
        acc_ref[1] = jnp.sum(z_ref[...])
        b = jnp.float32(nsteps) * jnp.float32(xb.shape[1])
        out_ref[0, 0] = acc_ref[0] / b + (acc_ref[1] / b) * (_EPS + acc_ref[2])


def kernel(x, delta, y, z):
    B, C = x.shape
    D = delta.shape[1]
    grid = B // _BB

    xt = x.T
    dt = delta.T

    out = pl.pallas_call(
        _body,
        grid=(grid,),
        in_specs=[
            pl.BlockSpec((C, _BB), lambda i: (0, i)),
            pl.BlockSpec((B,), lambda i: (0,)),
            pl.BlockSpec((D, _BB), lambda i: (0, i)),
            pl.BlockSpec((B,), lambda i: (0,)),
        ],
        out_specs=pl.BlockSpec(
            (1, 1), lambda i: (0, 0), memory_space=pltpu.SMEM
        ),
        out_shape=jax.ShapeDtypeStruct((1, 1), jnp.float32),
        scratch_shapes=[pltpu.SMEM((3,), jnp.float32)],
    )(xt, y.astype(jnp.int32), dt, z)
    return out[0, 0]


# speedup vs baseline: 1.4547x; 1.0594x over previous
import jax
import jax.numpy as jnp
from jax import lax
from jax.experimental import pallas as pl
from jax.experimental.pallas import tpu as pltpu

_K = 0.05
_EPS = 0.3


def _body(x_ref, y_ref, d_ref, z_ref, out_ref, acc_ref):
    step = pl.program_id(0)
    nsteps = pl.num_programs(0)

    @pl.when(step == 0)
    def _init():
        acc_ref[0] = 0.0
        acc_ref[1] = 0.0
        acc_ref[2] = -jnp.inf

    xb = x_ref[...]
    yb = y_ref[...][None, :]
    rows = lax.broadcasted_iota(jnp.int32, xb.shape, 0)
    onehot = rows == yb
    target = jnp.sum(jnp.where(onehot, xb, 0.0), axis=0)
    rest_max = jnp.max(jnp.where(onehot, -jnp.inf, xb), axis=0)
    relu_sum = jnp.sum(jnp.maximum(target - rest_max + _K, 0.0))

    acc_ref[0] += relu_sum
    acc_ref[1] += jnp.sum(z_ref[...])
    acc_ref[2] = jnp.maximum(acc_ref[2], jnp.max(d_ref[...]))

    @pl.when(step == nsteps - 1)
    def _fini():
        b = jnp.float32(nsteps) * jnp.float32(xb.shape[1])
        out_ref[0, 0] = acc_ref[0] / b + (acc_ref[1] / b) * (_EPS + acc_ref[2])


def kernel(x, delta, y, z):
    B, C = x.shape
    D = delta.shape[1]
    BB = 2048
    grid = B // BB

    xt = x.T
    dt = delta.T

    out = pl.pallas_call(
        _body,
        grid=(grid,),
        in_specs=[
            pl.BlockSpec((C, BB), lambda i: (0, i)),
            pl.BlockSpec((BB,), lambda i: (i,)),
            pl.BlockSpec((D, BB), lambda i: (0, i)),
            pl.BlockSpec((BB,), lambda i: (i,)),
        ],
        out_specs=pl.BlockSpec(
            (1, 1), lambda i: (0, 0), memory_space=pltpu.SMEM
        ),
        out_shape=jax.ShapeDtypeStruct((1, 1), jnp.float32),
        scratch_shapes=[pltpu.SMEM((3,), jnp.float32)],
    )(xt, y.astype(jnp.int32), dt, z)
    return out[0, 0]
